# Initial kernel scaffold; baseline (speedup 1.0000x reference)
import jax
import jax.numpy as jnp
from jax.experimental import pallas as pl


def kernel(query_embedding, selected_role_embedding, selected_llm_embedding, selected_edge_index, selected_edge_embedding, W_enc, b_enc, W_gcn, b_gcn, W_head, b_head):
    raise NotImplementedError("write your pallas kernel here")


# speedup vs baseline: 76.8428x; 76.8428x over previous
import functools

import jax
import jax.numpy as jnp
from jax import lax
from jax.experimental import pallas as pl
from jax.experimental.pallas import tpu as pltpu
from jax.experimental.pallas import tpu_sc as plsc

_NC = 2
_NS = 16
_LANES = 16


def _deg_body(dst_hbm, ew_hbm, zeros_hbm, out_hbm, dst_v, ew_v, shared):
    c = lax.axis_index("c")
    s = lax.axis_index("s")
    ec = dst_v.shape[0]
    base = (c * _NS + s) * ec
    pltpu.sync_copy(dst_hbm.at[pl.ds(base, ec)], dst_v)
    pltpu.sync_copy(ew_hbm.at[pl.ds(base, ec)], ew_v)

    @pl.when(s == 0)
    def _():
        pltpu.sync_copy(zeros_hbm, shared)

    plsc.subcore_barrier()
    pltpu.sync_copy(ew_v, shared.at[dst_v], add=True)
    plsc.subcore_barrier()

    @pl.when(s == 0)
    def _():
        pltpu.sync_copy(shared, out_hbm.at[c])


def _edge_body(src_hbm, dst_hbm, ew_hbm, u_hbm, v_hbm, out_hbm,
               src_v, dst_v, ew_v, u_tab, v_tab, acc_v):
    c = lax.axis_index("c")
    s = lax.axis_index("s")
    ec = src_v.shape[0]
    wid = c * _NS + s
    base = wid * ec
    pltpu.sync_copy(u_hbm, u_tab)
    pltpu.sync_copy(v_hbm, v_tab)
    pltpu.sync_copy(src_hbm.at[pl.ds(base, ec)], src_v)
    pltpu.sync_copy(dst_hbm.at[pl.ds(base, ec)], dst_v)
    pltpu.sync_copy(ew_hbm.at[pl.ds(base, ec)], ew_v)

    def body(i, acc):
        off = i * _LANES
        s16 = src_v[pl.ds(off, _LANES)]
        d16 = dst_v[pl.ds(off, _LANES)]
        e16 = ew_v[pl.ds(off, _LANES)]
        us = plsc.load_gather(u_tab, [s16])
        vd = plsc.load_gather(v_tab, [d16])
        return acc + e16 * us * vd

    acc = lax.fori_loop(0, ec // _LANES, body,
                        jnp.zeros((_LANES,), jnp.float32))
    acc_v[...] = acc
    pltpu.sync_copy(acc_v, out_hbm.at[wid])


def _dense_body(q_ref, role_ref, llm_ref, wenc_ref, benc_ref, wgcn_ref,
                bgcn_ref, whead_ref, bhead_ref, deg0_ref, deg1_ref,
                u_ref, v_ref, sself_ref, bias_ref):
    i = pl.program_id(0)
    d = q_ref.shape[1]
    w1 = wenc_ref[0:d, :]
    w2 = wenc_ref[d:2 * d, :]
    w3 = wenc_ref[2 * d:3 * d, :]
    cvec = (jnp.dot(q_ref[...], w3, preferred_element_type=jnp.float32)
            + benc_ref[...][None, :])
    rql = (jnp.dot(role_ref[...], w1, preferred_element_type=jnp.float32)
           + jnp.dot(llm_ref[...], w2, preferred_element_type=jnp.float32)
           + cvec)
    nrm = jnp.sqrt(jnp.sum(rql * rql, axis=1, keepdims=True))
    rqln = rql / jnp.maximum(nrm, 1e-12)
    wc = jnp.dot(wgcn_ref[...], whead_ref[...],
                 preferred_element_type=jnp.float32)
    y = jnp.dot(rqln, wc, preferred_element_type=jnp.float32)
    deg = deg0_ref[...] + deg1_ref[...] + 1.0
    dinv = lax.rsqrt(deg)
    u_ref[...] = y * dinv
    v_ref[...] = dinv
    part = jnp.sum(y * dinv * dinv)

    @pl.when(i == 0)
    def _():
        sself_ref[...] = jnp.zeros_like(sself_ref)
        bias_ref[...] = jnp.reshape(
            jnp.sum(bgcn_ref[...] * whead_ref[...][:, 0])
            + jnp.sum(bhead_ref[...]), (1, 1))

    sself_ref[...] += jnp.reshape(part, (1, 1))


def kernel(query_embedding, selected_role_embedding, selected_llm_embedding,
           selected_edge_index, selected_edge_embedding,
           W_enc, b_enc, W_gcn, b_gcn, W_head, b_head):
    n, d = selected_role_embedding.shape
    e = selected_edge_index.shape[1]
    nw = _NC * _NS
    assert e % (nw * _LANES) == 0
    ec = e // nw

    src = selected_edge_index[0]
    dst = selected_edge_index[1]
    ew = selected_edge_embedding
    zeros_n = jnp.zeros((n,), jnp.float32)

    mesh = plsc.VectorSubcoreMesh(core_axis_name="c", subcore_axis_name="s",
                                  num_cores=_NC, num_subcores=_NS)

    deg_kernel = functools.partial(
        pl.kernel,
        mesh=mesh,
        out_type=jax.ShapeDtypeStruct((_NC, n), jnp.float32),
        scratch_types=[
            pltpu.VMEM((ec,), jnp.int32),
            pltpu.VMEM((ec,), jnp.float32),
            pltpu.VMEM_SHARED((n,), jnp.float32),
        ],
    )(_deg_body)
    degp = deg_kernel(dst, ew, zeros_n)

    deg0 = degp[0].reshape(n, 1)
    deg1 = degp[1].reshape(n, 1)

    rows = 1000 if n % 1000 == 0 else n
    grid = (n // rows,)
    dspec = pl.BlockSpec((rows, d), lambda i: (i, 0))
    cspec = pl.BlockSpec((rows, 1), lambda i: (i, 0))
    u2, v2, sself, bias = pl.pallas_call(
        _dense_body,
        grid=grid,
        in_specs=[
            pl.BlockSpec((1, d), lambda i: (0, 0)),
            dspec,
            dspec,
            pl.BlockSpec((3 * d, d), lambda i: (0, 0)),
            pl.BlockSpec((d,), lambda i: (0,)),
            pl.BlockSpec((d, d), lambda i: (0, 0)),
            pl.BlockSpec((d,), lambda i: (0,)),
            pl.BlockSpec((d, 1), lambda i: (0, 0)),
            pl.BlockSpec((1,), lambda i: (0,)),
            cspec,
            cspec,
        ],
        out_specs=[
            cspec,
            cspec,
            pl.BlockSpec((1, 1), lambda i: (0, 0)),
            pl.BlockSpec((1, 1), lambda i: (0, 0)),
        ],
        out_shape=[
            jax.ShapeDtypeStruct((n, 1), jnp.float32),
            jax.ShapeDtypeStruct((n, 1), jnp.float32),
            jax.ShapeDtypeStruct((1, 1), jnp.float32),
            jax.ShapeDtypeStruct((1, 1), jnp.float32),
        ],
    )(query_embedding, selected_role_embedding, selected_llm_embedding,
      W_enc, b_enc, W_gcn, b_gcn, W_head, b_head, deg0, deg1)

    u = u2.reshape(n)
    v = v2.reshape(n)

    edge_kernel = functools.partial(
        pl.kernel,
        mesh=mesh,
        compiler_params=pltpu.CompilerParams(needs_layout_passes=False),
        out_type=jax.ShapeDtypeStruct((nw, _LANES), jnp.float32),
        scratch_types=[
            pltpu.VMEM((ec,), jnp.int32),
            pltpu.VMEM((ec,), jnp.int32),
            pltpu.VMEM((ec,), jnp.float32),
            pltpu.VMEM((n,), jnp.float32),
            pltpu.VMEM((n,), jnp.float32),
            pltpu.VMEM((_LANES,), jnp.float32),
        ],
    )(_edge_body)
    parts = edge_kernel(src, dst, ew, u, v)

    state = (jnp.sum(parts) + sself[0, 0]) / n + bias[0, 0]
    return jnp.reshape(state, (1,))


# speedup vs baseline: 88.9146x; 1.1571x over previous
import functools

import jax
import jax.numpy as jnp
from jax import lax
from jax.experimental import pallas as pl
from jax.experimental.pallas import tpu as pltpu
from jax.experimental.pallas import tpu_sc as plsc

_NC = 2
_NS = 16
_LANES = 16


def _rsqrt16(x):
    i = plsc.bitcast(x, jnp.int32)
    i = jnp.full((_LANES,), 0x5F3759DF, jnp.int32) - lax.shift_right_logical(
        i, jnp.full((_LANES,), 1, jnp.int32))
    r = plsc.bitcast(i, jnp.float32)
    half = x * 0.5
    for _ in range(4):
        r = r * (1.5 - half * r * r)
    return r


def _sc_body(src_hbm, dst_hbm, ew_hbm, y_hbm, zeros_hbm, out_hbm,
             dst_v, ew_v, src_v, y_v, deg_t, u_tab, v_tab, acc_v, shared):
    c = lax.axis_index("c")
    s = lax.axis_index("s")
    epc = dst_v.shape[0]
    ec = src_v.shape[0]
    n = y_v.shape[0]

    pltpu.sync_copy(dst_hbm.at[pl.ds(s * epc, epc)], dst_v)
    pltpu.sync_copy(ew_hbm.at[pl.ds(s * epc, epc)], ew_v)
    pltpu.sync_copy(y_hbm, y_v)
    wid = s * _NC + c
    pltpu.sync_copy(src_hbm.at[pl.ds(wid * ec, ec)], src_v)

    @pl.when(s == 0)
    def _():
        pltpu.sync_copy(zeros_hbm, shared)

    plsc.subcore_barrier()
    pltpu.sync_copy(ew_v, shared.at[dst_v], add=True)
    plsc.subcore_barrier()
    pltpu.sync_copy(shared, deg_t)

    def tab_body(i, carry):
        off = i * _LANES
        d16 = deg_t[pl.ds(off, _LANES)] + 1.0
        r = _rsqrt16(d16)
        y16 = y_v[pl.ds(off, _LANES)]
        u_tab[pl.ds(off, _LANES)] = y16 * r
        v_tab[pl.ds(off, _LANES)] = r
        return carry

    lax.fori_loop(0, n // _LANES, tab_body, 0)

    loc = c * ec

    def edge_body(i, acc):
        off = i * _LANES
        s16 = src_v[pl.ds(off, _LANES)]
        d16 = dst_v[pl.ds(loc + off, _LANES)]
        e16 = ew_v[pl.ds(loc + off, _LANES)]
        us = plsc.load_gather(u_tab, [s16])
        vd = plsc.load_gather(v_tab, [d16])
        return acc + e16 * us * vd

    acc = lax.fori_loop(0, ec // _LANES, edge_body,
                        jnp.zeros((_LANES,), jnp.float32))

    @pl.when((c == 0) & (s == 0))
    def _():
        def self_body(i, a):
            off = i * _LANES
            return a + u_tab[pl.ds(off, _LANES)] * v_tab[pl.ds(off, _LANES)]

        acc_v[...] = lax.fori_loop(0, n // _LANES, self_body, acc)

    @pl.when((c != 0) | (s != 0))
    def _():
        acc_v[...] = acc

    pltpu.sync_copy(acc_v, out_hbm.at[wid])


def _dense_body(q_ref, role_ref, llm_ref, wenc_ref, benc_ref, wgcn_ref,
                bgcn_ref, whead_ref, bhead_ref, y_ref, bias_ref):
    d = q_ref.shape[1]
    w1 = wenc_ref[0:d, :]
    w2 = wenc_ref[d:2 * d, :]
    w3 = wenc_ref[2 * d:3 * d, :]
    cvec = (jnp.dot(q_ref[...], w3, preferred_element_type=jnp.float32)
            + benc_ref[...][None, :])
    rql = (jnp.dot(role_ref[...], w1, preferred_element_type=jnp.float32)
           + jnp.dot(llm_ref[...], w2, preferred_element_type=jnp.float32)
           + cvec)
    nrm = jnp.sqrt(jnp.sum(rql * rql, axis=1, keepdims=True))
    wc = jnp.dot(wgcn_ref[...], whead_ref[...],
                 preferred_element_type=jnp.float32)
    t = jnp.dot(rql, wc, preferred_element_type=jnp.float32)
    y = t / jnp.maximum(nrm, 1e-12)
    y_ref[...] = jnp.reshape(y, (y_ref.shape[0],))
    bias_ref[...] = jnp.reshape(
        jnp.sum(bgcn_ref[...] * whead_ref[...][:, 0])
        + jnp.sum(bhead_ref[...]), (1, 1))


def kernel(query_embedding, selected_role_embedding, selected_llm_embedding,
           selected_edge_index, selected_edge_embedding,
           W_enc, b_enc, W_gcn, b_gcn, W_head, b_head):
    n, d = selected_role_embedding.shape
    e = selected_edge_index.shape[1]
    nw = _NC * _NS
    assert e % (nw * _LANES) == 0 and n % _LANES == 0
    ec = e // nw
    epc = e // _NS

    src = selected_edge_index[0]
    dst = selected_edge_index[1]
    ew = selected_edge_embedding
    zeros_n = jnp.zeros((n,), jnp.float32)

    y1d, bias = pl.pallas_call(
        _dense_body,
        out_shape=[
            jax.ShapeDtypeStruct((n,), jnp.float32),
            jax.ShapeDtypeStruct((1, 1), jnp.float32),
        ],
    )(query_embedding, selected_role_embedding, selected_llm_embedding,
      W_enc, b_enc, W_gcn, b_gcn, W_head, b_head)

    mesh = plsc.VectorSubcoreMesh(core_axis_name="c", subcore_axis_name="s",
                                  num_cores=_NC, num_subcores=_NS)
    sc_kernel = functools.partial(
        pl.kernel,
        mesh=mesh,
        compiler_params=pltpu.CompilerParams(needs_layout_passes=False),
        out_type=jax.ShapeDtypeStruct((nw, _LANES), jnp.float32),
        scratch_types=[
            pltpu.VMEM((epc,), jnp.int32),
            pltpu.VMEM((epc,), jnp.float32),
            pltpu.VMEM((ec,), jnp.int32),
            pltpu.VMEM((n,), jnp.float32),
            pltpu.VMEM((n,), jnp.float32),
            pltpu.VMEM((n,), jnp.float32),
            pltpu.VMEM((n,), jnp.float32),
            pltpu.VMEM((_LANES,), jnp.float32),
            pltpu.VMEM_SHARED((n,), jnp.float32),
        ],
    )(_sc_body)
    parts = sc_kernel(src, dst, ew, y1d, zeros_n)

    state = jnp.sum(parts) / n + bias[0, 0]
    return jnp.reshape(state, (1,))


# speedup vs baseline: 100.8230x; 1.1339x over previous
import functools

import jax
import jax.numpy as jnp
from jax import lax
from jax.experimental import pallas as pl
from jax.experimental.pallas import tpu as pltpu
from jax.experimental.pallas import tpu_sc as plsc

_NC = 2
_NS = 16
_LANES = 16


def _rsqrt16(x):
    i = plsc.bitcast(x, jnp.int32)
    i = jnp.full((_LANES,), 0x5F3759DF, jnp.int32) - lax.shift_right_logical(
        i, jnp.full((_LANES,), 1, jnp.int32))
    r = plsc.bitcast(i, jnp.float32)
    half = x * 0.5
    for _ in range(4):
        r = r * (1.5 - half * r * r)
    return r


def _sc_body(ei_hbm, ew_hbm, y_hbm, zeros_hbm, out_hbm,
             ei_v, ew_v, dst_f, eix_v, ewx_v, dstx_f, y_v, deg_t, u_tab,
             v_tab, acc_v, shared):
    c = lax.axis_index("c")
    s = lax.axis_index("s")
    epc = ei_v.shape[1]
    nx = eix_v.shape[1]
    n = y_v.shape[0]
    e_main = epc * _NS
    n_extra = (ei_hbm.shape[1] - e_main) // nx

    pltpu.sync_copy(ei_hbm.at[:, pl.ds(s * epc, epc)], ei_v)
    pltpu.sync_copy(ew_hbm.at[pl.ds(s * epc, epc)], ew_v)
    pltpu.sync_copy(y_hbm, y_v)
    def flat_body(i, carry):
        off = i * _LANES
        dst_f[pl.ds(off, _LANES)] = ei_v[1, pl.ds(off, _LANES)]
        return carry

    lax.fori_loop(0, epc // _LANES, flat_body, 0)

    @pl.when(s < n_extra)
    def _():
        pltpu.sync_copy(ei_hbm.at[:, pl.ds(e_main + s * nx, nx)], eix_v)
        pltpu.sync_copy(ew_hbm.at[pl.ds(e_main + s * nx, nx)], ewx_v)

        def flatx_body(i, carry):
            off = i * _LANES
            dstx_f[pl.ds(off, _LANES)] = eix_v[1, pl.ds(off, _LANES)]
            return carry

        lax.fori_loop(0, nx // _LANES, flatx_body, 0)

    @pl.when(s == 0)
    def _():
        pltpu.sync_copy(zeros_hbm, shared)

    plsc.subcore_barrier()
    pltpu.sync_copy(ew_v, shared.at[dst_f], add=True)

    @pl.when(s < n_extra)
    def _():
        pltpu.sync_copy(ewx_v, shared.at[dstx_f], add=True)

    plsc.subcore_barrier()
    pltpu.sync_copy(shared, deg_t)

    def tab_body(i, carry):
        off = i * _LANES
        d16 = deg_t[pl.ds(off, _LANES)] + 1.0
        r = _rsqrt16(d16)
        y16 = y_v[pl.ds(off, _LANES)]
        u_tab[pl.ds(off, _LANES)] = y16 * r
        v_tab[pl.ds(off, _LANES)] = r
        return carry

    lax.fori_loop(0, n // _LANES, tab_body, 0)

    half = epc // 2
    loc = c * half

    def edge_body(i, acc):
        off = loc + i * _LANES
        s16 = ei_v[0, pl.ds(off, _LANES)]
        d16 = ei_v[1, pl.ds(off, _LANES)]
        e16 = ew_v[pl.ds(off, _LANES)]
        us = plsc.load_gather(u_tab, [s16])
        vd = plsc.load_gather(v_tab, [d16])
        return acc + e16 * us * vd

    acc = lax.fori_loop(0, half // _LANES, edge_body,
                        jnp.zeros((_LANES,), jnp.float32))

    halfx = nx // 2
    locx = c * halfx

    def edgex_body(i, acc):
        off = locx + i * _LANES
        s16 = eix_v[0, pl.ds(off, _LANES)]
        d16 = eix_v[1, pl.ds(off, _LANES)]
        e16 = ewx_v[pl.ds(off, _LANES)]
        us = plsc.load_gather(u_tab, [s16])
        vd = plsc.load_gather(v_tab, [d16])
        return acc + e16 * us * vd

    acc = lax.cond(s < n_extra,
                   lambda a: lax.fori_loop(0, halfx // _LANES, edgex_body, a),
                   lambda a: a, acc)

    @pl.when((c == 0) & (s == 0))
    def _():
        def self_body(i, a):
            off = i * _LANES
            return a + u_tab[pl.ds(off, _LANES)] * v_tab[pl.ds(off, _LANES)]

        acc_v[...] = lax.fori_loop(0, n // _LANES, self_body, acc)

    @pl.when((c != 0) | (s != 0))
    def _():
        acc_v[...] = acc

    wid = s * _NC + c
    pltpu.sync_copy(acc_v, out_hbm.at[wid])


def _dense_body(q_ref, role_ref, llm_ref, wenc_ref, benc_ref, wgcn_ref,
                bgcn_ref, whead_ref, bhead_ref, y_ref, bias_ref):
    d = q_ref.shape[1]
    w1 = wenc_ref[0:d, :]
    w2 = wenc_ref[d:2 * d, :]
    w3 = wenc_ref[2 * d:3 * d, :]
    cvec = (jnp.dot(q_ref[...], w3, preferred_element_type=jnp.float32)
            + benc_ref[...][None, :])
    rql = (jnp.dot(role_ref[...], w1, preferred_element_type=jnp.float32)
           + jnp.dot(llm_ref[...], w2, preferred_element_type=jnp.float32)
           + cvec)
    nrm = jnp.sqrt(jnp.sum(rql * rql, axis=1, keepdims=True))
    wc = jnp.dot(wgcn_ref[...], whead_ref[...],
                 preferred_element_type=jnp.float32)
    t = jnp.dot(rql, wc, preferred_element_type=jnp.float32)
    y = t / jnp.maximum(nrm, 1e-12)
    y_ref[...] = jnp.reshape(y, (y_ref.shape[0],))
    bias_ref[...] = jnp.reshape(
        jnp.sum(bgcn_ref[...] * whead_ref[...][:, 0])
        + jnp.sum(bhead_ref[...]), (1, 1))


def kernel(query_embedding, selected_role_embedding, selected_llm_embedding,
           selected_edge_index, selected_edge_embedding,
           W_enc, b_enc, W_gcn, b_gcn, W_head, b_head):
    n, d = selected_role_embedding.shape
    e = selected_edge_index.shape[1]
    nw = _NC * _NS
    assert n % _LANES == 0
    epc = (e // (_NS * 128)) * 128
    nx = 128
    n_extra = (e - epc * _NS) // nx
    assert epc * _NS + n_extra * nx == e and n_extra <= _NS
    assert epc % (2 * _LANES) == 0 and nx % (2 * _LANES) == 0

    ew = selected_edge_embedding
    zeros_n = jnp.zeros((n,), jnp.float32)

    y1d, bias = pl.pallas_call(
        _dense_body,
        out_shape=[
            jax.ShapeDtypeStruct((n,), jnp.float32),
            jax.ShapeDtypeStruct((1, 1), jnp.float32),
        ],
    )(query_embedding, selected_role_embedding, selected_llm_embedding,
      W_enc, b_enc, W_gcn, b_gcn, W_head, b_head)

    mesh = plsc.VectorSubcoreMesh(core_axis_name="c", subcore_axis_name="s",
                                  num_cores=_NC, num_subcores=_NS)
    sc_kernel = functools.partial(
        pl.kernel,
        mesh=mesh,
        compiler_params=pltpu.CompilerParams(needs_layout_passes=False),
        out_type=jax.ShapeDtypeStruct((nw, _LANES), jnp.float32),
        scratch_types=[
            pltpu.VMEM((2, epc), jnp.int32),
            pltpu.VMEM((epc,), jnp.float32),
            pltpu.VMEM((epc,), jnp.int32),
            pltpu.VMEM((2, nx), jnp.int32),
            pltpu.VMEM((nx,), jnp.float32),
            pltpu.VMEM((nx,), jnp.int32),
            pltpu.VMEM((n,), jnp.float32),
            pltpu.VMEM((n,), jnp.float32),
            pltpu.VMEM((n,), jnp.float32),
            pltpu.VMEM((n,), jnp.float32),
            pltpu.VMEM((_LANES,), jnp.float32),
            pltpu.VMEM_SHARED((n,), jnp.float32),
        ],
    )(_sc_body)
    parts = sc_kernel(selected_edge_index, ew, y1d, zeros_n)

    state = jnp.sum(parts) / n + bias[0, 0]
    return jnp.reshape(state, (1,))


# speedup vs baseline: 100.9670x; 1.0014x over previous
import functools

import jax
import jax.numpy as jnp
from jax import lax
from jax.experimental import pallas as pl
from jax.experimental.pallas import tpu as pltpu
from jax.experimental.pallas import tpu_sc as plsc

_NC = 2
_NS = 16
_LANES = 16
_TILE = 128


def _rsqrt16(x):
    i = plsc.bitcast(x, jnp.int32)
    i = jnp.full((_LANES,), 0x5F3759DF, jnp.int32) - lax.shift_right_logical(
        i, jnp.full((_LANES,), 1, jnp.int32))
    r = plsc.bitcast(i, jnp.float32)
    half = x * 0.5
    for _ in range(4):
        r = r * (1.5 - half * r * r)
    return r


def _deg_body(ei_hbm, ew_hbm, out_hbm, ei_v, ew_v, dst_f, eix_v, ewx_v,
              dstx_f, z_v, shared):
    c = lax.axis_index("c")
    s = lax.axis_index("s")
    epc = ei_v.shape[1]
    nx = eix_v.shape[1]
    n = z_v.shape[0]
    e_half = ei_hbm.shape[1] // _NC
    e_main = epc * _NS
    n_extra = (e_half - e_main) // nx
    sc_base = c * e_half

    pltpu.sync_copy(ei_hbm.at[:, pl.ds(sc_base + s * epc, epc)], ei_v)
    pltpu.sync_copy(ew_hbm.at[pl.ds(sc_base + s * epc, epc)], ew_v)

    def flat_body(i, carry):
        off = i * _LANES
        dst_f[pl.ds(off, _LANES)] = ei_v[1, pl.ds(off, _LANES)]
        return carry

    lax.fori_loop(0, epc // _LANES, flat_body, 0)

    @pl.when(s < n_extra)
    def _():
        xb = sc_base + e_main + s * nx
        pltpu.sync_copy(ei_hbm.at[:, pl.ds(xb, nx)], eix_v)
        pltpu.sync_copy(ew_hbm.at[pl.ds(xb, nx)], ewx_v)

        def flatx_body(i, carry):
            off = i * _LANES
            dstx_f[pl.ds(off, _LANES)] = eix_v[1, pl.ds(off, _LANES)]
            return carry

        lax.fori_loop(0, nx // _LANES, flatx_body, 0)

    @pl.when(s == 0)
    def _():
        def zero_body(i, carry):
            z_v[pl.ds(i * _LANES, _LANES)] = jnp.zeros((_LANES,), jnp.float32)
            return carry

        lax.fori_loop(0, n // _LANES, zero_body, 0)
        pltpu.sync_copy(z_v, shared)

    plsc.subcore_barrier()
    pltpu.sync_copy(ew_v, shared.at[dst_f], add=True)

    @pl.when(s < n_extra)
    def _():
        pltpu.sync_copy(ewx_v, shared.at[dstx_f], add=True)

    plsc.subcore_barrier()

    @pl.when(s == 0)
    def _():
        pltpu.sync_copy(shared, out_hbm.at[c])


def _edge_body(ei_hbm, ew_hbm, y_hbm, deg_hbm, out_hbm,
               ei_v, ew_v, eix_v, ewx_v, y_v, d2_v, u_tab, v_tab, acc_v):
    c = lax.axis_index("c")
    s = lax.axis_index("s")
    epc = ei_v.shape[1]
    nx = eix_v.shape[1]
    n = y_v.shape[0]
    n_pad = u_tab.shape[0]
    e_main = epc * _NS
    n_extra = (ei_hbm.shape[1] - e_main) // nx

    pltpu.sync_copy(ei_hbm.at[:, pl.ds(s * epc, epc)], ei_v)
    pltpu.sync_copy(ew_hbm.at[pl.ds(s * epc, epc)], ew_v)
    pltpu.sync_copy(y_hbm.at[pl.ds(0, n)], y_v)
    pltpu.sync_copy(deg_hbm, d2_v)

    @pl.when(s < n_extra)
    def _():
        pltpu.sync_copy(ei_hbm.at[:, pl.ds(e_main + s * nx, nx)], eix_v)
        pltpu.sync_copy(ew_hbm.at[pl.ds(e_main + s * nx, nx)], ewx_v)

    def tab_body(i, carry):
        off = i * _LANES
        d16 = d2_v[0, pl.ds(off, _LANES)] + d2_v[1, pl.ds(off, _LANES)] + 1.0
        r = _rsqrt16(d16)
        y16 = y_v[pl.ds(off, _LANES)]
        u_tab[pl.ds(off, _LANES)] = y16 * r
        v_tab[pl.ds(off, _LANES)] = r
        return carry

    lax.fori_loop(0, n // _LANES, tab_body, 0)

    def pad_body(i, carry):
        off = n + i * _LANES
        u_tab[pl.ds(off, _LANES)] = jnp.zeros((_LANES,), jnp.float32)
        v_tab[pl.ds(off, _LANES)] = jnp.zeros((_LANES,), jnp.float32)
        return carry

    lax.fori_loop(0, (n_pad - n) // _LANES, pad_body, 0)

    half = epc // 2
    loc = c * half

    def edge_body(i, acc):
        off = loc + i * _LANES
        s16 = ei_v[0, pl.ds(off, _LANES)]
        d16 = ei_v[1, pl.ds(off, _LANES)]
        e16 = ew_v[pl.ds(off, _LANES)]
        us = plsc.load_gather(u_tab, [s16])
        vd = plsc.load_gather(v_tab, [d16])
        return acc + e16 * us * vd

    acc = lax.fori_loop(0, half // _LANES, edge_body,
                        jnp.zeros((_LANES,), jnp.float32))

    halfx = nx // 2
    locx = c * halfx

    def edgex_body(i, acc):
        off = locx + i * _LANES
        s16 = eix_v[0, pl.ds(off, _LANES)]
        d16 = eix_v[1, pl.ds(off, _LANES)]
        e16 = ewx_v[pl.ds(off, _LANES)]
        us = plsc.load_gather(u_tab, [s16])
        vd = plsc.load_gather(v_tab, [d16])
        return acc + e16 * us * vd

    acc = lax.cond(s < n_extra,
                   lambda a: lax.fori_loop(0, halfx // _LANES, edgex_body, a),
                   lambda a: a, acc)

    wid = s * _NC + c
    per = n_pad // (_NC * _NS)
    sbase = wid * per

    def self_body(i, a):
        off = sbase + i * _LANES
        return a + u_tab[pl.ds(off, _LANES)] * v_tab[pl.ds(off, _LANES)]

    acc = lax.fori_loop(0, per // _LANES, self_body, acc)

    acc_v[...] = acc
    pltpu.sync_copy(acc_v, out_hbm.at[wid])


def _dense_body(q_ref, role_ref, llm_ref, wenc_ref, benc_ref, wgcn_ref,
                bgcn_ref, whead_ref, bhead_ref, y_ref, bias_ref):
    i = pl.program_id(0)
    d = q_ref.shape[1]
    w1 = wenc_ref[0:d, :]
    w2 = wenc_ref[d:2 * d, :]
    w3 = wenc_ref[2 * d:3 * d, :]
    cvec = (jnp.dot(q_ref[...], w3, preferred_element_type=jnp.float32)
            + benc_ref[...][None, :])
    rql = (jnp.dot(role_ref[...], w1, preferred_element_type=jnp.float32)
           + jnp.dot(llm_ref[...], w2, preferred_element_type=jnp.float32)
           + cvec)
    nrm = jnp.sqrt(jnp.sum(rql * rql, axis=1, keepdims=True))
    wc = jnp.dot(wgcn_ref[...], whead_ref[...],
                 preferred_element_type=jnp.float32)
    t = jnp.dot(rql, wc, preferred_element_type=jnp.float32)
    y = t / jnp.maximum(nrm, 1e-12)
    y_ref[...] = jnp.reshape(y, (y_ref.shape[0],))

    @pl.when(i == 0)
    def _():
        bias_ref[...] = jnp.reshape(
            jnp.sum(bgcn_ref[...] * whead_ref[...][:, 0])
            + jnp.sum(bhead_ref[...]), (1, 1))


def kernel(query_embedding, selected_role_embedding, selected_llm_embedding,
           selected_edge_index, selected_edge_embedding,
           W_enc, b_enc, W_gcn, b_gcn, W_head, b_head):
    n, d = selected_role_embedding.shape
    e = selected_edge_index.shape[1]
    nw = _NC * _NS
    assert n % _LANES == 0
    ew = selected_edge_embedding
    ei = selected_edge_index

    e_half = e // _NC
    epc_a = (e_half // (_NS * _TILE)) * _TILE
    nxa = _TILE
    n_extra_a = (e_half - epc_a * _NS) // nxa
    assert epc_a * _NS + n_extra_a * nxa == e_half and n_extra_a <= _NS
    assert epc_a % _LANES == 0

    mesh = plsc.VectorSubcoreMesh(core_axis_name="c", subcore_axis_name="s",
                                  num_cores=_NC, num_subcores=_NS)
    deg_kernel = functools.partial(
        pl.kernel,
        mesh=mesh,
        compiler_params=pltpu.CompilerParams(needs_layout_passes=False),
        out_type=jax.ShapeDtypeStruct((_NC, n), jnp.float32),
        scratch_types=[
            pltpu.VMEM((2, epc_a), jnp.int32),
            pltpu.VMEM((epc_a,), jnp.float32),
            pltpu.VMEM((epc_a,), jnp.int32),
            pltpu.VMEM((2, nxa), jnp.int32),
            pltpu.VMEM((nxa,), jnp.float32),
            pltpu.VMEM((nxa,), jnp.int32),
            pltpu.VMEM((n,), jnp.float32),
            pltpu.VMEM_SHARED((n,), jnp.float32),
        ],
    )(_deg_body)
    deg2n = deg_kernel(ei, ew)

    rows = 1024
    grid = (pl.cdiv(n, rows),)
    n_pad = grid[0] * rows
    assert n_pad % nw == 0 and (n_pad // nw) % _LANES == 0
    dspec = pl.BlockSpec((rows, d), lambda i: (i, 0))
    y1d, bias = pl.pallas_call(
        _dense_body,
        grid=grid,
        in_specs=[
            pl.BlockSpec((1, d), lambda i: (0, 0)),
            dspec,
            dspec,
            pl.BlockSpec((3 * d, d), lambda i: (0, 0)),
            pl.BlockSpec((d,), lambda i: (0,)),
            pl.BlockSpec((d, d), lambda i: (0, 0)),
            pl.BlockSpec((d,), lambda i: (0,)),
            pl.BlockSpec((d, 1), lambda i: (0, 0)),
            pl.BlockSpec((1,), lambda i: (0,)),
        ],
        out_specs=[
            pl.BlockSpec((rows,), lambda i: (i,)),
            pl.BlockSpec((1, 1), lambda i: (0, 0)),
        ],
        out_shape=[
            jax.ShapeDtypeStruct((n_pad,), jnp.float32),
            jax.ShapeDtypeStruct((1, 1), jnp.float32),
        ],
    )(query_embedding, selected_role_embedding, selected_llm_embedding,
      W_enc, b_enc, W_gcn, b_gcn, W_head, b_head)

    epc_b = (e // (_NS * _TILE)) * _TILE
    nxb = _TILE
    n_extra_b = (e - epc_b * _NS) // nxb
    assert epc_b * _NS + n_extra_b * nxb == e and n_extra_b <= _NS
    assert epc_b % (2 * _LANES) == 0 and nxb % (2 * _LANES) == 0

    edge_kernel = functools.partial(
        pl.kernel,
        mesh=mesh,
        compiler_params=pltpu.CompilerParams(needs_layout_passes=False),
        out_type=jax.ShapeDtypeStruct((nw, _LANES), jnp.float32),
        scratch_types=[
            pltpu.VMEM((2, epc_b), jnp.int32),
            pltpu.VMEM((epc_b,), jnp.float32),
            pltpu.VMEM((2, nxb), jnp.int32),
            pltpu.VMEM((nxb,), jnp.float32),
            pltpu.VMEM((n,), jnp.float32),
            pltpu.VMEM((2, n), jnp.float32),
            pltpu.VMEM((n_pad,), jnp.float32),
            pltpu.VMEM((n_pad,), jnp.float32),
            pltpu.VMEM((_LANES,), jnp.float32),
        ],
    )(_edge_body)
    parts = edge_kernel(ei, ew, y1d, deg2n)

    state = jnp.sum(parts) / n + bias[0, 0]
    return jnp.reshape(state, (1,))
